# baseline (speedup 1.0000x reference)
import jax
import jax.numpy as jnp
from jax.experimental import pallas as pl
from jax.experimental.pallas import tpu as pltpu

_N_INPUT = 28 * 28
_N_RES = 1000
_T = 25
_BETA = 0.95
_TH = 1.0

_K_PAD = 896
_N_PAD = 1024
_B_BLK = 128
_B_HALF = _B_BLK // 2
_UNROLL = 24


def _lsm_body(x_ref, win_ref, wrec_ref, o_ref):
    icA = jnp.dot(x_ref[:_B_HALF], win_ref[...],
                  preferred_element_type=jnp.float32)
    icB = jnp.dot(x_ref[_B_HALF:], win_ref[...],
                  preferred_element_type=jnp.float32)

    memA = icA
    memB = icB
    ssA = (memA - _TH > 0).astype(jnp.bfloat16)
    ssB = (memB - _TH > 0).astype(jnp.bfloat16)

    def one_step(memA, ssA, memB, ssB):
        spkA = (memA - _TH > 0).astype(jnp.float32)
        spkB = (memB - _TH > 0).astype(jnp.float32)
        recA = jnp.dot(spkA, wrec_ref[...], preferred_element_type=jnp.float32)
        recB = jnp.dot(spkB, wrec_ref[...], preferred_element_type=jnp.float32)
        memA = _BETA * memA + (icA + recA) - spkA * _TH
        memB = _BETA * memB + (icB + recB) - spkB * _TH
        ssA = ssA + (memA - _TH > 0).astype(jnp.bfloat16)
        ssB = ssB + (memB - _TH > 0).astype(jnp.bfloat16)
        return memA, ssA, memB, ssB

    def stepn(_, c):
        for _i in range(_UNROLL):
            c = one_step(*c)
        return c

    memA, ssA, memB, ssB = jax.lax.fori_loop(
        0, (_T - 1) // _UNROLL, stepn, (memA, ssA, memB, ssB))
    o_ref[:_B_HALF] = ssA.astype(jnp.float32) * (1.0 / _T)
    o_ref[_B_HALF:] = ssB.astype(jnp.float32) * (1.0 / _T)


def kernel(x, W_in, W_rec):
    B = x.shape[0]
    x_p = jnp.pad(x, ((0, 0), (0, _K_PAD - _N_INPUT)))
    win_t = jnp.pad(W_in.T, ((0, _K_PAD - _N_INPUT), (0, _N_PAD - _N_RES)))
    wrec_t = jnp.pad(W_rec.T, ((0, _N_PAD - _N_RES), (0, _N_PAD - _N_RES)))

    out = pl.pallas_call(
        _lsm_body,
        out_shape=jax.ShapeDtypeStruct((B, _N_PAD), jnp.float32),
        grid=(B // _B_BLK,),
        in_specs=[
            pl.BlockSpec((_B_BLK, _K_PAD), lambda b: (b, 0)),
            pl.BlockSpec((_K_PAD, _N_PAD), lambda b: (0, 0)),
            pl.BlockSpec((_N_PAD, _N_PAD), lambda b: (0, 0)),
        ],
        out_specs=pl.BlockSpec((_B_BLK, _N_PAD), lambda b: (b, 0)),
        compiler_params=pltpu.CompilerParams(
            dimension_semantics=("parallel",),
        ),
        name="lsm_fused",
    )(x_p, win_t, wrec_t)
    return out[:, :_N_RES]


# speedup vs baseline: 2.0027x; 2.0027x over previous
import jax
import jax.numpy as jnp
from jax.experimental import pallas as pl
from jax.experimental.pallas import tpu as pltpu

_N_INPUT = 28 * 28
_N_RES = 1000
_T = 25
_BETA = 0.95
_TH = 1.0

_K_PAD = 896
_N_PAD = 1024
_B_BLK = 256
_B_HALF = _B_BLK // 2
_UNROLL = 24


def _lsm_body(x_ref, win_ref, wrec_ref, o_ref):
    icA = jnp.dot(x_ref[:_B_HALF], win_ref[...],
                  preferred_element_type=jnp.float32)
    icB = jnp.dot(x_ref[_B_HALF:], win_ref[...],
                  preferred_element_type=jnp.float32)

    memA = icA
    memB = icB
    ssA = (memA - _TH > 0).astype(jnp.bfloat16)
    ssB = (memB - _TH > 0).astype(jnp.bfloat16)

    def one_step(memA, ssA, memB, ssB):
        spkA = (memA - _TH > 0).astype(jnp.float32)
        spkB = (memB - _TH > 0).astype(jnp.float32)
        recA = jnp.dot(spkA, wrec_ref[...], preferred_element_type=jnp.float32)
        recB = jnp.dot(spkB, wrec_ref[...], preferred_element_type=jnp.float32)
        memA = _BETA * memA + (icA + recA) - spkA * _TH
        memB = _BETA * memB + (icB + recB) - spkB * _TH
        ssA = ssA + (memA - _TH > 0).astype(jnp.bfloat16)
        ssB = ssB + (memB - _TH > 0).astype(jnp.bfloat16)
        return memA, ssA, memB, ssB

    def stepn(_, c):
        for _i in range(_UNROLL):
            c = one_step(*c)
        return c

    memA, ssA, memB, ssB = jax.lax.fori_loop(
        0, (_T - 1) // _UNROLL, stepn, (memA, ssA, memB, ssB))
    o_ref[:_B_HALF] = ssA.astype(jnp.float32) * (1.0 / _T)
    o_ref[_B_HALF:] = ssB.astype(jnp.float32) * (1.0 / _T)


def kernel(x, W_in, W_rec):
    B = x.shape[0]
    x_p = jnp.pad(x, ((0, 0), (0, _K_PAD - _N_INPUT)))
    win_t = jnp.pad(W_in.T, ((0, _K_PAD - _N_INPUT), (0, _N_PAD - _N_RES)))
    wrec_t = jnp.pad(W_rec.T, ((0, _N_PAD - _N_RES), (0, _N_PAD - _N_RES)))

    out = pl.pallas_call(
        _lsm_body,
        out_shape=jax.ShapeDtypeStruct((B, _N_PAD), jnp.float32),
        grid=(B // _B_BLK,),
        in_specs=[
            pl.BlockSpec((_B_BLK, _K_PAD), lambda b: (b, 0)),
            pl.BlockSpec((_K_PAD, _N_PAD), lambda b: (0, 0)),
            pl.BlockSpec((_N_PAD, _N_PAD), lambda b: (0, 0)),
        ],
        out_specs=pl.BlockSpec((_B_BLK, _N_PAD), lambda b: (b, 0)),
        compiler_params=pltpu.CompilerParams(
            dimension_semantics=("parallel",),
        ),
        name="lsm_fused",
    )(x_p, win_t, wrec_t)
    return out[:, :_N_RES]
